# baseline (speedup 1.0000x reference)
import functools

import jax
import jax.numpy as jnp
from jax import lax
from jax.experimental import pallas as pl
from jax.experimental.pallas import tpu as pltpu
from jax.experimental.pallas import tpu_sc as plsc

BS = 1024
MM = 1024
DD = 1024
DE = 64
KK = 32
SCALE = DE ** (-0.5)

NEG_INF = float("-inf")


def _body(x_ref, y_ref, emk_ref, emv_ref, ems_ref, wq_ref, bq_ref,
          wqc_ref, bqc_ref, wo_ref, bo_ref, out_ref, *, bblk):
    xb = x_ref[...]
    yb = y_ref[...]
    cat16 = jnp.concatenate([xb, yb], axis=1).astype(jnp.bfloat16)
    w16 = wq_ref[...].astype(jnp.bfloat16)
    q = jnp.dot(cat16, w16, preferred_element_type=jnp.float32)
    q = q + bq_ref[...]
    norm = jnp.sqrt(jnp.sum(q * q, axis=1, keepdims=True))
    q = q / (norm + 1e-8)
    q16 = q.astype(jnp.bfloat16)

    emk16 = emk_ref[...].astype(jnp.bfloat16)
    scores = lax.dot_general(q16, emk16, (((1,), (2,)), ((0,), (0,))),
                             preferred_element_type=jnp.float32)
    scores = jnp.where(ems_ref[...] > 0, scores, NEG_INF)

    iota = lax.broadcasted_iota(jnp.int32, (bblk, MM), 1)
    vals, idxs = [], []
    s = scores
    for _ in range(KK):
        m = jnp.max(s, axis=1, keepdims=True)
        cand = jnp.min(jnp.where(s == m, iota, jnp.int32(MM)),
                       axis=1, keepdims=True)
        vals.append(m)
        idxs.append(cand)
        s = jnp.where(iota == cand, NEG_INF, s)
    topk_vals = jnp.concatenate(vals, axis=1)
    topk_idx = jnp.concatenate(idxs, axis=1)

    onehot = (topk_idx[:, :, None] == iota[:, None, :]).astype(jnp.bfloat16)
    vtop = lax.dot_general(onehot, emv_ref[...].astype(jnp.bfloat16),
                           ((( 2,), (1,)), ((0,), (0,))),
                           preferred_element_type=jnp.float32)

    qc = jnp.dot(xb.astype(jnp.bfloat16), wqc_ref[...].astype(jnp.bfloat16),
                 preferred_element_type=jnp.float32) + bqc_ref[...]
    qc = qc.astype(jnp.bfloat16).astype(jnp.float32)
    vtop16 = vtop.astype(jnp.bfloat16).astype(jnp.float32)
    logits = jnp.sum(vtop16 * qc[:, None, :], axis=2) * SCALE
    valid = topk_vals != NEG_INF
    a = jnp.where(valid, logits, -1e30)
    mx = jnp.max(a, axis=1, keepdims=True)
    e = jnp.exp(a - mx) * valid.astype(jnp.float32)
    ssum = jnp.sum(e, axis=1, keepdims=True)
    p = e / jnp.where(ssum == 0, 1.0, ssum)
    out = jnp.sum(vtop16 * p[:, :, None], axis=1)
    out_ref[...] = (jnp.dot(out.astype(jnp.bfloat16),
                            wo_ref[...].astype(jnp.bfloat16),
                            preferred_element_type=jnp.float32) + bo_ref[...])


def _run(x, y_wm, em_K, em_V, em_S, wq_t, bq, wqc_t, bqc, wo_t, bo,
         *, bblk=8, interpret=False):
    grid = (BS // bblk,)
    return pl.pallas_call(
        functools.partial(_body, bblk=bblk),
        grid=grid,
        in_specs=[
            pl.BlockSpec((bblk, DD), lambda i: (i, 0)),
            pl.BlockSpec((bblk, DD), lambda i: (i, 0)),
            pl.BlockSpec((bblk, MM, DE), lambda i: (i, 0, 0)),
            pl.BlockSpec((bblk, MM, DE), lambda i: (i, 0, 0)),
            pl.BlockSpec((bblk, MM), lambda i: (i, 0)),
            pl.BlockSpec((2 * DD, DE), lambda i: (0, 0)),
            pl.BlockSpec((1, DE), lambda i: (0, 0)),
            pl.BlockSpec((DD, DE), lambda i: (0, 0)),
            pl.BlockSpec((1, DE), lambda i: (0, 0)),
            pl.BlockSpec((DE, DD), lambda i: (0, 0)),
            pl.BlockSpec((1, DD), lambda i: (0, 0)),
        ],
        out_specs=pl.BlockSpec((bblk, DD), lambda i: (i, 0)),
        out_shape=jax.ShapeDtypeStruct((BS, DD), jnp.float32),
        interpret=interpret,
    )(x, y_wm, em_K, em_V, em_S, wq_t, bq, wqc_t, bqc, wo_t, bo)


def kernel(x, y_wm, em_K, em_V, em_S, W_q_em, b_q_em,
           W_q_cross, b_q_cross, W_o_cross, b_o_cross):
    wq_t = W_q_em.T
    wqc_t = W_q_cross.T
    wo_t = W_o_cross.T
    bq = b_q_em.reshape(1, DE)
    bqc = b_q_cross.reshape(1, DE)
    bo = b_o_cross.reshape(1, DD)
    return _run(x, y_wm, em_K, em_V, em_S, wq_t, bq, wqc_t, bqc, wo_t, bo)


# speedup vs baseline: 1.8123x; 1.8123x over previous
import functools

import jax
import jax.numpy as jnp
from jax import lax
from jax.experimental import pallas as pl
from jax.experimental.pallas import tpu as pltpu
from jax.experimental.pallas import tpu_sc as plsc

BS = 1024
MM = 1024
DD = 1024
DE = 64
KK = 32
SCALE = DE ** (-0.5)

NEG_INF = float("-inf")


def _body(x_ref, y_ref, emk_ref, emv_ref, ems_ref, wq_ref, bq_ref,
          wqc_ref, bqc_ref, wo_ref, bo_ref, out_ref, *, bblk):
    xb = x_ref[...]
    yb = y_ref[...]
    cat16 = jnp.concatenate([xb, yb], axis=1).astype(jnp.bfloat16)
    w16 = wq_ref[...].astype(jnp.bfloat16)
    q = jnp.dot(cat16, w16, preferred_element_type=jnp.float32)
    q = q + bq_ref[...]
    norm = jnp.sqrt(jnp.sum(q * q, axis=1, keepdims=True))
    q = q / (norm + 1e-8)
    q16 = q.astype(jnp.bfloat16)

    emk16 = emk_ref[...].astype(jnp.bfloat16)
    scores = lax.dot_general(q16, emk16, (((1,), (2,)), ((0,), (0,))),
                             preferred_element_type=jnp.float32)
    scores = jnp.where(ems_ref[...] > 0, scores, NEG_INF)
    finite = scores != NEG_INF

    s = scores
    for _ in range(KK):
        m = jnp.max(s, axis=1, keepdims=True)
        s = jnp.where(s == m, NEG_INF, s)
    sel = finite & (s == NEG_INF)

    qc = jnp.dot(xb.astype(jnp.bfloat16), wqc_ref[...].astype(jnp.bfloat16),
                 preferred_element_type=jnp.float32) + bqc_ref[...]
    qc16 = qc.astype(jnp.bfloat16)
    emv16 = emv_ref[...].astype(jnp.bfloat16)
    logits = lax.dot_general(qc16, emv16, (((1,), (2,)), ((0,), (0,))),
                             preferred_element_type=jnp.float32) * SCALE
    a = jnp.where(sel, logits, -1e30)
    mx = jnp.max(a, axis=1, keepdims=True)
    e = jnp.exp(a - mx) * sel.astype(jnp.float32)
    ssum = jnp.sum(e, axis=1, keepdims=True)
    p = e / jnp.where(ssum == 0, 1.0, ssum)
    out = lax.dot_general(p.astype(jnp.bfloat16), emv16,
                          (((1,), (1,)), ((0,), (0,))),
                          preferred_element_type=jnp.float32)
    out_ref[...] = (jnp.dot(out.astype(jnp.bfloat16),
                            wo_ref[...].astype(jnp.bfloat16),
                            preferred_element_type=jnp.float32) + bo_ref[...])


def _run(x, y_wm, em_K, em_V, em_S, wq_t, bq, wqc_t, bqc, wo_t, bo,
         *, bblk=16, interpret=False):
    grid = (BS // bblk,)
    return pl.pallas_call(
        functools.partial(_body, bblk=bblk),
        grid=grid,
        in_specs=[
            pl.BlockSpec((bblk, DD), lambda i: (i, 0)),
            pl.BlockSpec((bblk, DD), lambda i: (i, 0)),
            pl.BlockSpec((bblk, MM, DE), lambda i: (i, 0, 0)),
            pl.BlockSpec((bblk, MM, DE), lambda i: (i, 0, 0)),
            pl.BlockSpec((bblk, MM), lambda i: (i, 0)),
            pl.BlockSpec((2 * DD, DE), lambda i: (0, 0)),
            pl.BlockSpec((1, DE), lambda i: (0, 0)),
            pl.BlockSpec((DD, DE), lambda i: (0, 0)),
            pl.BlockSpec((1, DE), lambda i: (0, 0)),
            pl.BlockSpec((DE, DD), lambda i: (0, 0)),
            pl.BlockSpec((1, DD), lambda i: (0, 0)),
        ],
        out_specs=pl.BlockSpec((bblk, DD), lambda i: (i, 0)),
        out_shape=jax.ShapeDtypeStruct((BS, DD), jnp.float32),
        interpret=interpret,
    )(x, y_wm, em_K, em_V, em_S, wq_t, bq, wqc_t, bqc, wo_t, bo)


def kernel(x, y_wm, em_K, em_V, em_S, W_q_em, b_q_em,
           W_q_cross, b_q_cross, W_o_cross, b_o_cross):
    wq_t = W_q_em.T
    wqc_t = W_q_cross.T
    wo_t = W_o_cross.T
    bq = b_q_em.reshape(1, DE)
    bqc = b_q_cross.reshape(1, DE)
    bo = b_o_cross.reshape(1, DD)
    return _run(x, y_wm, em_K, em_V, em_S, wq_t, bq, wqc_t, bqc, wo_t, bo)
